# Initial kernel scaffold; baseline (speedup 1.0000x reference)
import jax
import jax.numpy as jnp
from jax.experimental import pallas as pl


def kernel(x, W_enc, b_enc, W_dec, b_dec):
    raise NotImplementedError("write your pallas kernel here")


# speedup vs baseline: 1.3345x; 1.3345x over previous
import functools

import jax
import jax.numpy as jnp
from jax.experimental import pallas as pl
from jax.experimental.pallas import tpu as pltpu

_ACT_DIM = 2048
_DICT = 32768
_K = 64
_B = 8

_BD_E = 2048
_BD_D = 2048

_I32_MIN = -2147483648
_I32_MAX = 2147483647


def _sortable_key(z):
    bits = jax.lax.bitcast_convert_type(z, jnp.int32)
    return jnp.where(bits >= 0, bits, bits ^ jnp.int32(0x7FFFFFFF))


def _encode_kernel(x_ref, w_ref, b_ref, sz_ref, z_scr, key_scr):
    i = pl.program_id(0)
    nb = pl.num_programs(0)
    zblk = jax.lax.dot_general(
        x_ref[...], w_ref[...], (((1,), (1,)), ((), ())),
        preferred_element_type=jnp.float32) + b_ref[...]
    z_scr[:, pl.ds(i * _BD_E, _BD_E)] = zblk
    key_scr[:, pl.ds(i * _BD_E, _BD_E)] = _sortable_key(zblk)

    @pl.when(i == nb - 1)
    def _finish():
        key = key_scr[...]

        def body(_, carry):
            lo, hi = carry
            mid = (lo >> 1) + (hi >> 1) + (lo & hi & 1)
            cnt = jnp.sum((key >= mid).astype(jnp.int32), axis=1,
                          keepdims=True)
            ge = cnt >= _K
            return jnp.where(ge, mid, lo), jnp.where(ge, hi, mid)

        lo0 = jnp.full((_B, 1), jnp.iinfo(jnp.int32).min, jnp.int32)
        hi0 = jnp.full((_B, 1), jnp.iinfo(jnp.int32).max, jnp.int32)
        thr, _ = jax.lax.fori_loop(0, 32, body, (lo0, hi0))
        sz_ref[...] = jnp.where(key >= thr, z_scr[...], 0.0)


def _decode_kernel(sz_ref, w_ref, b_ref, out_ref, acc):
    i = pl.program_id(0)
    nb = pl.num_programs(0)

    @pl.when(i == 0)
    def _init():
        acc[...] = jnp.zeros_like(acc)

    acc[...] += jax.lax.dot_general(
        sz_ref[...], w_ref[...], (((1,), (1,)), ((), ())),
        preferred_element_type=jnp.float32)

    @pl.when(i == nb - 1)
    def _finish():
        out_ref[...] = acc[...] + b_ref[...]


@functools.partial(jax.jit, static_argnames=())
def kernel(x, W_enc, b_enc, W_dec, b_dec):
    b_enc2 = b_enc.reshape(1, _DICT)
    b_dec2 = b_dec.reshape(1, _ACT_DIM)

    nb_e = _DICT // _BD_E
    sparse_z = pl.pallas_call(
        _encode_kernel,
        grid=(nb_e,),
        in_specs=[
            pl.BlockSpec((_B, _ACT_DIM), lambda i: (0, 0)),
            pl.BlockSpec((_BD_E, _ACT_DIM), lambda i: (i, 0)),
            pl.BlockSpec((1, _BD_E), lambda i: (0, i)),
        ],
        out_specs=pl.BlockSpec((_B, _DICT), lambda i: (0, 0)),
        out_shape=jax.ShapeDtypeStruct((_B, _DICT), jnp.float32),
        scratch_shapes=[
            pltpu.VMEM((_B, _DICT), jnp.float32),
            pltpu.VMEM((_B, _DICT), jnp.int32),
        ],
    )(x, W_enc, b_enc2)

    nb_d = _DICT // _BD_D
    x_hat = pl.pallas_call(
        _decode_kernel,
        grid=(nb_d,),
        in_specs=[
            pl.BlockSpec((_B, _BD_D), lambda i: (0, i)),
            pl.BlockSpec((_ACT_DIM, _BD_D), lambda i: (0, i)),
            pl.BlockSpec((1, _ACT_DIM), lambda i: (0, 0)),
        ],
        out_specs=pl.BlockSpec((_B, _ACT_DIM), lambda i: (0, 0)),
        out_shape=jax.ShapeDtypeStruct((_B, _ACT_DIM), jnp.float32),
        scratch_shapes=[pltpu.VMEM((_B, _ACT_DIM), jnp.float32)],
    )(sparse_z, W_dec, b_dec2)

    return (x_hat, sparse_z)
